# baseline (speedup 1.0000x reference)
import functools

import jax
import jax.numpy as jnp
from jax import lax
from jax.experimental import pallas as pl
from jax.experimental.pallas import tpu as pltpu
from jax.experimental.pallas import tpu_sc as plsc

_K = 512
_D = 32


def _dist_argmin_body(zr_ref, wt_ref, idx_ref):
    n = zr_ref.shape[0]
    acc = None
    for c in range(4):
        y = []
        for s in range(8):
            d = 8 * c + s
            diff = zr_ref[:, d:d + 1] - wt_ref[d:d + 1, :]
            y.append(diff * diff)
        u0 = y[4] + y[0]
        u1 = y[5] + y[1]
        u2 = y[6] + y[2]
        u3 = y[7] + y[3]
        t = (u3 + u1) + (u2 + u0)
        acc = t if acc is None else acc + t
    dist = jnp.sqrt(acc)
    minv = jnp.min(dist, axis=1, keepdims=True)
    ids = lax.broadcasted_iota(jnp.int32, (n, _K), 1)
    idx_ref[...] = jnp.min(jnp.where(dist == minv, ids, _K), axis=1,
                           keepdims=True)


def _tc_argmin(zr, wt):
    n = zr.shape[0]
    nb = 256
    return pl.pallas_call(
        _dist_argmin_body,
        grid=(n // nb,),
        in_specs=[
            pl.BlockSpec((nb, _D), lambda i: (i, 0)),
            pl.BlockSpec((_D, _K), lambda i: (0, 0)),
        ],
        out_specs=pl.BlockSpec((nb, 1), lambda i: (i, 0)),
        out_shape=jax.ShapeDtypeStruct((n, 1), jnp.int32),
        compiler_params=pltpu.CompilerParams(
            dimension_semantics=("arbitrary",)),
    )(zr, wt)


def _sc_gather(table, idx):
    info = plsc.get_sparse_core_info()
    nc, ns = info.num_cores, info.num_subcores
    nw = nc * ns
    b = idx.shape[0]
    b_per_w = b // nw
    mesh = plsc.VectorSubcoreMesh(core_axis_name="c", subcore_axis_name="s")

    @functools.partial(
        pl.kernel, mesh=mesh,
        compiler_params=pltpu.CompilerParams(use_tc_tiling_on_sc=False),
        out_type=jax.ShapeDtypeStruct((b, _D), jnp.float32),
        scratch_types=[
            pltpu.VMEM((b_per_w,), jnp.int32),
            pltpu.VMEM((b_per_w, _D), jnp.float32),
            pltpu.SemaphoreType.DMA,
        ],
    )
    def k(table_hbm, idx_hbm, out_hbm, idx_v, rows_v, sem):
        wid = lax.axis_index("s") * nc + lax.axis_index("c")
        base = wid * b_per_w
        pltpu.sync_copy(idx_hbm.at[pl.ds(base, b_per_w)], idx_v)
        pltpu.async_copy(table_hbm.at[idx_v], rows_v, sem).wait()
        pltpu.sync_copy(rows_v, out_hbm.at[pl.ds(base, b_per_w)])

    return k(table, idx)


def kernel(z, W):
    bs, cdim, t, h, w = z.shape
    zr = jnp.transpose(z, (0, 2, 3, 4, 1)).reshape(-1, cdim)
    idx = _tc_argmin(zr, W.T).reshape(-1)
    q_r = _sc_gather(W, idx)
    quantized = jnp.transpose(q_r.reshape(bs, t, h, w, cdim), (0, 4, 1, 2, 3))
    straight_through = z + lax.stop_gradient(quantized - z)
    encoding_indices = idx.reshape(bs, t, h, w)
    return (quantized, straight_through, encoding_indices)


# speedup vs baseline: 1.0261x; 1.0261x over previous
import functools

import jax
import jax.numpy as jnp
from jax import lax
from jax.experimental import pallas as pl
from jax.experimental.pallas import tpu as pltpu
from jax.experimental.pallas import tpu_sc as plsc

_K = 512
_D = 32


def _dist_argmin_body(zr_ref, wt_ref, idx_ref):
    n = zr_ref.shape[0]
    acc = None
    for c in range(4):
        y = []
        for s in range(8):
            d = 8 * c + s
            diff = zr_ref[:, d:d + 1] - wt_ref[d:d + 1, :]
            y.append(diff * diff)
        u0 = y[4] + y[0]
        u1 = y[5] + y[1]
        u2 = y[6] + y[2]
        u3 = y[7] + y[3]
        t = (u3 + u1) + (u2 + u0)
        acc = t if acc is None else acc + t
    dist = jnp.sqrt(acc)
    minv = jnp.min(dist, axis=1, keepdims=True)
    ids = lax.broadcasted_iota(jnp.int32, (n, _K), 1)
    idx_ref[...] = jnp.min(jnp.where(dist == minv, ids, _K), axis=1,
                           keepdims=True)


def _tc_argmin(zr, wt):
    n = zr.shape[0]
    nb = 1024
    return pl.pallas_call(
        _dist_argmin_body,
        grid=(n // nb,),
        in_specs=[
            pl.BlockSpec((nb, _D), lambda i: (i, 0)),
            pl.BlockSpec((_D, _K), lambda i: (0, 0)),
        ],
        out_specs=pl.BlockSpec((nb, 1), lambda i: (i, 0)),
        out_shape=jax.ShapeDtypeStruct((n, 1), jnp.int32),
        compiler_params=pltpu.CompilerParams(
            dimension_semantics=("arbitrary",)),
    )(zr, wt)


def _sc_gather(table, idx):
    info = plsc.get_sparse_core_info()
    nc, ns = info.num_cores, info.num_subcores
    nw = nc * ns
    b = idx.shape[0]
    b_per_w = b // nw
    mesh = plsc.VectorSubcoreMesh(core_axis_name="c", subcore_axis_name="s")

    @functools.partial(
        pl.kernel, mesh=mesh,
        compiler_params=pltpu.CompilerParams(use_tc_tiling_on_sc=False),
        out_type=jax.ShapeDtypeStruct((b, _D), jnp.float32),
        scratch_types=[
            pltpu.VMEM((b_per_w,), jnp.int32),
            pltpu.VMEM((b_per_w, _D), jnp.float32),
            pltpu.SemaphoreType.DMA,
        ],
    )
    def k(table_hbm, idx_hbm, out_hbm, idx_v, rows_v, sem):
        wid = lax.axis_index("s") * nc + lax.axis_index("c")
        base = wid * b_per_w
        pltpu.sync_copy(idx_hbm.at[pl.ds(base, b_per_w)], idx_v)
        pltpu.async_copy(table_hbm.at[idx_v], rows_v, sem).wait()
        pltpu.sync_copy(rows_v, out_hbm.at[pl.ds(base, b_per_w)])

    return k(table, idx)


def kernel(z, W):
    bs, cdim, t, h, w = z.shape
    zr = jnp.transpose(z, (0, 2, 3, 4, 1)).reshape(-1, cdim)
    idx = _tc_argmin(zr, W.T).reshape(-1)
    q_r = _sc_gather(W, idx)
    quantized = jnp.transpose(q_r.reshape(bs, t, h, w, cdim), (0, 4, 1, 2, 3))
    straight_through = z + lax.stop_gradient(quantized - z)
    encoding_indices = idx.reshape(bs, t, h, w)
    return (quantized, straight_through, encoding_indices)


# speedup vs baseline: 1.0645x; 1.0374x over previous
import functools

import jax
import jax.numpy as jnp
from jax import lax
from jax.experimental import pallas as pl
from jax.experimental.pallas import tpu as pltpu
from jax.experimental.pallas import tpu_sc as plsc

_K = 512
_D = 32


def _dist_argmin_body(zr_ref, wt_ref, idx_ref):
    n = zr_ref.shape[0]
    acc = None
    for c in range(4):
        y = []
        for s in range(8):
            d = 8 * c + s
            diff = zr_ref[:, d:d + 1] - wt_ref[d:d + 1, :]
            y.append(diff * diff)
        u0 = y[4] + y[0]
        u1 = y[5] + y[1]
        u2 = y[6] + y[2]
        u3 = y[7] + y[3]
        t = (u3 + u1) + (u2 + u0)
        acc = t if acc is None else acc + t
    dist = jnp.sqrt(acc)
    minv = jnp.min(dist, axis=1, keepdims=True)
    ids = lax.broadcasted_iota(jnp.int32, (n, _K), 1)
    idx_ref[...] = jnp.min(jnp.where(dist == minv, ids, _K), axis=1,
                           keepdims=True)


def _tc_argmin(zr, wt):
    n = zr.shape[0]
    nb = 512
    return pl.pallas_call(
        _dist_argmin_body,
        grid=(n // nb,),
        in_specs=[
            pl.BlockSpec((nb, _D), lambda i: (i, 0)),
            pl.BlockSpec((_D, _K), lambda i: (0, 0)),
        ],
        out_specs=pl.BlockSpec((nb, 1), lambda i: (i, 0)),
        out_shape=jax.ShapeDtypeStruct((n, 1), jnp.int32),
        compiler_params=pltpu.CompilerParams(
            dimension_semantics=("arbitrary",),
            allow_input_fusion=(True, True)),
    )(zr, wt)


def _sc_gather(table, idx):
    info = plsc.get_sparse_core_info()
    nc, ns = info.num_cores, info.num_subcores
    nw = nc * ns
    b = idx.shape[0]
    b_per_w = b // nw
    mesh = plsc.VectorSubcoreMesh(core_axis_name="c", subcore_axis_name="s")

    @functools.partial(
        pl.kernel, mesh=mesh,
        compiler_params=pltpu.CompilerParams(use_tc_tiling_on_sc=False),
        out_type=jax.ShapeDtypeStruct((b, _D), jnp.float32),
        scratch_types=[
            pltpu.VMEM((b_per_w,), jnp.int32),
            pltpu.VMEM((b_per_w, _D), jnp.float32),
            pltpu.SemaphoreType.DMA,
        ],
    )
    def k(table_hbm, idx_hbm, out_hbm, idx_v, rows_v, sem):
        wid = lax.axis_index("s") * nc + lax.axis_index("c")
        base = wid * b_per_w
        pltpu.sync_copy(idx_hbm.at[pl.ds(base, b_per_w)], idx_v)
        pltpu.async_copy(table_hbm.at[idx_v], rows_v, sem).wait()
        pltpu.sync_copy(rows_v, out_hbm.at[pl.ds(base, b_per_w)])

    return k(table, idx)


def kernel(z, W):
    bs, cdim, t, h, w = z.shape
    zr = jnp.transpose(z, (0, 2, 3, 4, 1)).reshape(-1, cdim)
    idx = _tc_argmin(zr, W.T).reshape(-1)
    q_r = _sc_gather(W, idx)
    quantized = jnp.transpose(q_r.reshape(bs, t, h, w, cdim), (0, 4, 1, 2, 3))
    straight_through = z + lax.stop_gradient(quantized - z)
    encoding_indices = idx.reshape(bs, t, h, w)
    return (quantized, straight_through, encoding_indices)


# speedup vs baseline: 1.0655x; 1.0010x over previous
import functools

import jax
import jax.numpy as jnp
from jax import lax
from jax.experimental import pallas as pl
from jax.experimental.pallas import tpu as pltpu
from jax.experimental.pallas import tpu_sc as plsc

_K = 512
_D = 32


def _dist_argmin_body(zr_ref, wt_ref, idx_ref):
    n = zr_ref.shape[0]
    acc = None
    for c in range(4):
        y = []
        for s in range(8):
            d = 8 * c + s
            diff = zr_ref[:, d:d + 1] - wt_ref[d:d + 1, :]
            y.append(diff * diff)
        u0 = y[4] + y[0]
        u1 = y[5] + y[1]
        u2 = y[6] + y[2]
        u3 = y[7] + y[3]
        t = (u3 + u1) + (u2 + u0)
        acc = t if acc is None else acc + t
    dist = jnp.sqrt(acc)
    minv = jnp.min(dist, axis=1, keepdims=True)
    ids = lax.broadcasted_iota(jnp.int32, (n, _K), 1)
    idx_ref[...] = jnp.min(jnp.where(dist == minv, ids, _K), axis=1,
                           keepdims=True)


def _tc_argmin(zr, wt):
    n = zr.shape[0]
    nb = 512
    return pl.pallas_call(
        _dist_argmin_body,
        grid=(n // nb,),
        in_specs=[
            pl.BlockSpec((nb, _D), lambda i: (i, 0)),
            pl.BlockSpec((_D, _K), lambda i: (0, 0)),
        ],
        out_specs=pl.BlockSpec((nb, 1), lambda i: (i, 0)),
        out_shape=jax.ShapeDtypeStruct((n, 1), jnp.int32),
        compiler_params=pltpu.CompilerParams(
            dimension_semantics=("arbitrary",),
            allow_input_fusion=(True, True)),
    )(zr, wt)


def _sc_gather(table, idx):
    info = plsc.get_sparse_core_info()
    nc, ns = info.num_cores, info.num_subcores
    nw = nc * ns
    b = idx.shape[0]
    b_per_w = b // nw
    mesh = plsc.VectorSubcoreMesh(core_axis_name="c", subcore_axis_name="s")

    @functools.partial(
        pl.kernel, mesh=mesh,
        compiler_params=pltpu.CompilerParams(use_tc_tiling_on_sc=False),
        out_type=jax.ShapeDtypeStruct((b, _D), jnp.float32),
        scratch_types=[
            pltpu.VMEM((b_per_w,), jnp.int32),
            pltpu.VMEM((b_per_w, _D), jnp.float32),
            pltpu.SemaphoreType.DMA,
        ],
    )
    def k(table_hbm, idx_hbm, out_hbm, idx_v, rows_v, sem):
        wid = lax.axis_index("s") * nc + lax.axis_index("c")
        base = wid * b_per_w
        pltpu.sync_copy(idx_hbm.at[pl.ds(base, b_per_w)], idx_v)
        pltpu.async_copy(table_hbm.at[idx_v], rows_v, sem).wait()
        pltpu.sync_copy(rows_v, out_hbm.at[pl.ds(base, b_per_w)])

    return k(table, idx)


def kernel(z, W):
    bs, cdim, t, h, w = z.shape
    n = t * h * w
    zr = jnp.transpose(z, (0, 2, 3, 4, 1)).reshape(-1, cdim)
    wt = W.T
    idxs = [_tc_argmin(zr[b * n:(b + 1) * n], wt).reshape(-1)
            for b in range(bs)]
    q_rs = [_sc_gather(W, ib) for ib in idxs]
    q_r = jnp.concatenate(q_rs, axis=0)
    idx = jnp.concatenate(idxs, axis=0)
    quantized = jnp.transpose(q_r.reshape(bs, t, h, w, cdim), (0, 4, 1, 2, 3))
    straight_through = z + lax.stop_gradient(quantized - z)
    encoding_indices = idx.reshape(bs, t, h, w)
    return (quantized, straight_through, encoding_indices)
